# Initial kernel scaffold; baseline (speedup 1.0000x reference)
import jax
import jax.numpy as jnp
from jax.experimental import pallas as pl


def kernel(x, token_embedding):
    raise NotImplementedError("write your pallas kernel here")


# speedup vs baseline: 1.6823x; 1.6823x over previous
import jax
import jax.numpy as jnp
from jax.experimental import pallas as pl

_BN = 512


def _add_body(x_ref, pos_ref, o_ref):
    o_ref[0] = x_ref[0] + pos_ref[...]


@jax.jit
def kernel(x, token_embedding):
    B, N, D = x.shape
    return pl.pallas_call(
        _add_body,
        grid=(N // _BN, B),
        in_specs=[
            pl.BlockSpec((1, _BN, D), lambda n, b: (b, n, 0)),
            pl.BlockSpec((_BN, D), lambda n, b: (n, 0)),
        ],
        out_specs=pl.BlockSpec((1, _BN, D), lambda n, b: (b, n, 0)),
        out_shape=jax.ShapeDtypeStruct((B, N, D), x.dtype),
    )(x, token_embedding)


# speedup vs baseline: 1.8549x; 1.1026x over previous
import jax
import jax.numpy as jnp
from jax.experimental import pallas as pl

_BN = 1024


def _add_body(x_ref, pos_ref, o_ref):
    o_ref[0] = x_ref[0] + pos_ref[...]


@jax.jit
def kernel(x, token_embedding):
    B, N, D = x.shape
    return pl.pallas_call(
        _add_body,
        grid=(N // _BN, B),
        in_specs=[
            pl.BlockSpec((1, _BN, D), lambda n, b: (b, n, 0)),
            pl.BlockSpec((_BN, D), lambda n, b: (n, 0)),
        ],
        out_specs=pl.BlockSpec((1, _BN, D), lambda n, b: (b, n, 0)),
        out_shape=jax.ShapeDtypeStruct((B, N, D), x.dtype),
    )(x, token_embedding)


# speedup vs baseline: 1.9686x; 1.0613x over previous
import jax
import jax.numpy as jnp
from jax.experimental import pallas as pl

_BN = 2048


def _add_body(x_ref, pos_ref, o_ref):
    o_ref[0] = x_ref[0] + pos_ref[...]


@jax.jit
def kernel(x, token_embedding):
    B, N, D = x.shape
    return pl.pallas_call(
        _add_body,
        grid=(N // _BN, B),
        in_specs=[
            pl.BlockSpec((1, _BN, D), lambda n, b: (b, n, 0)),
            pl.BlockSpec((_BN, D), lambda n, b: (n, 0)),
        ],
        out_specs=pl.BlockSpec((1, _BN, D), lambda n, b: (b, n, 0)),
        out_shape=jax.ShapeDtypeStruct((B, N, D), x.dtype),
    )(x, token_embedding)
